# baseline (speedup 1.0000x reference)
import functools

import jax
import jax.numpy as jnp
from jax import lax
from jax.experimental import pallas as pl
from jax.experimental.pallas import tpu as pltpu
from jax.experimental.pallas import tpu_sc as plsc

_Q = 512
_NB = 128
_SHIFT = 25
_NW = 32
_CH = 8


def _monotone_key(x):
    i = lax.bitcast_convert_type(x, jnp.int32)
    flip = lax.shift_right_arithmetic(i, 31) & jnp.int32(0x7FFFFFFF)
    return i ^ flip


def _sc_hist_body(x_hbm, out_hbm, xbuf, hbuf):
    nc = 2
    wid = lax.axis_index("s") * nc + lax.axis_index("c")
    rows_per_worker = 16384 // _NW
    n_chunks = rows_per_worker // 16

    zeros16 = jnp.zeros((16,), jnp.int32)
    ones16 = jnp.ones((16,), jnp.int32)
    lane_base = lax.iota(jnp.int32, 16) * _Q

    def chunk_body(ci, carry):
        row0 = wid * rows_per_worker + ci * 16
        pltpu.sync_copy(x_hbm.at[pl.ds(row0 * _Q, 16 * _Q)], xbuf)
        def zero_body(j, c):
            for u in range(8):
                hbuf[pl.ds((j * 8 + u) * 16, 16)] = zeros16
            return c
        carry = lax.fori_loop(0, _NB // 8, zero_body, carry)
        def hist_body(p, c):
            for u in range(8):
                xv = plsc.load_gather(xbuf, [lane_base + (p * 8 + u)])
                key = _monotone_key(xv)
                b = lax.shift_right_arithmetic(key, _SHIFT) + jnp.int32(_NB // 2)
                plsc.addupdate_scatter(hbuf, [b * 16 + lax.iota(jnp.int32, 16)],
                                       ones16)
            return c
        carry = lax.fori_loop(0, _Q // 8, hist_body, carry)
        def scan_body(j, run):
            for u in range(8):
                run = run + hbuf[pl.ds((j * 8 + u) * 16, 16)]
                hbuf[pl.ds((j * 8 + u) * 16, 16)] = run
            return run
        lax.fori_loop(0, _NB // 8, scan_body, zeros16)
        pltpu.sync_copy(hbuf, out_hbm.at[pl.ds(row0 * _NB, 16 * _NB)])
        return carry

    lax.fori_loop(0, n_chunks, chunk_body, jnp.int32(0))


def _sc_hist(x):
    B = x.shape[0]
    mesh = plsc.VectorSubcoreMesh(core_axis_name="c", subcore_axis_name="s")
    kern = functools.partial(
        pl.kernel,
        mesh=mesh,
        out_type=jax.ShapeDtypeStruct((B * _NB,), jnp.int32),
        scratch_types=[
            pltpu.VMEM((16 * _Q,), jnp.float32),
            pltpu.VMEM((_NB * 16,), jnp.int32),
        ],
        compiler_params=pltpu.CompilerParams(needs_layout_passes=False),
    )(_sc_hist_body)
    out = kern(x.reshape(B * _Q)).reshape(B // 16, _NB, 16)
    return out.transpose(0, 2, 1).reshape(B, _NB)


def _mlp_kernel(x_ref, w1t_ref, w2t_ref, w3t_ref, k_ref):
    xb = x_ref[...]
    h = jnp.maximum(jnp.dot(xb, w1t_ref[...], preferred_element_type=jnp.float32), 0.0)
    h = jnp.maximum(jnp.dot(h, w2t_ref[...], preferred_element_type=jnp.float32), 0.0)
    scores = jnp.dot(h, w3t_ref[...], preferred_element_type=jnp.float32)
    k_ref[...] = (jnp.argmax(scores, axis=-1).astype(jnp.int32) + 1)[:, None]


def _mask_kernel(x_ref, cdf_ref, k_ref, wct_ref, o_ref):
    xb = x_ref[...]
    R, Q = xb.shape
    k = k_ref[...]
    keys = _monotone_key(xb)

    P = cdf_ref[...]
    bstar = jnp.sum((P <= (Q - k)).astype(jnp.int32), axis=-1, keepdims=True)

    lo0 = (bstar - jnp.int32(_NB // 2)) << _SHIFT
    hi0 = jnp.where(bstar == jnp.int32(_NB - 1),
                    jnp.int32(2147483647), lo0 + jnp.int32(1 << _SHIFT))

    def body(carry):
        lo, hi = carry
        mid = (lax.shift_right_arithmetic(lo, 1)
               + lax.shift_right_arithmetic(hi, 1)
               + (lo & hi & jnp.int32(1)))
        cnt = jnp.sum((keys >= mid).astype(jnp.int32), axis=-1, keepdims=True)
        pred = cnt >= k
        lo = jnp.where(pred, mid, lo)
        hi = jnp.where(pred, hi, mid)
        return lo, hi

    carry = (lo0, hi0)
    for _ in range(_SHIFT):
        carry = body(carry)
    t = carry[0]

    gt = keys > t
    eq = keys == t
    g = jnp.sum(gt.astype(jnp.int32), axis=-1, keepdims=True)

    eq_f = eq.astype(jnp.float32)
    jj = lax.broadcasted_iota(jnp.int32, (Q, Q), 0)
    ii = lax.broadcasted_iota(jnp.int32, (Q, Q), 1)
    strict_lower = (jj < ii).astype(jnp.float32)
    prefix = jnp.dot(eq_f, strict_lower, preferred_element_type=jnp.float32)
    prefix = prefix.astype(jnp.int32)

    mask = gt | (eq & ((g + prefix) < k))
    o_ref[...] = jnp.dot(mask.astype(jnp.float32), wct_ref[...],
                         preferred_element_type=jnp.float32)


@jax.jit
def kernel(x, W1, W2, W3, Wc):
    B, Q = x.shape
    D = Wc.shape[0]
    R = 1024

    w1t = W1.T
    w2t = W2.T
    w3t = W3.T
    wct = Wc.T

    cdf = _sc_hist(x)

    kvals = pl.pallas_call(
        _mlp_kernel,
        grid=(B // R,),
        in_specs=[
            pl.BlockSpec((R, Q), lambda i: (i, 0)),
            pl.BlockSpec((Q, 2 * Q), lambda i: (0, 0)),
            pl.BlockSpec((2 * Q, Q), lambda i: (0, 0)),
            pl.BlockSpec((Q, Q), lambda i: (0, 0)),
        ],
        out_specs=pl.BlockSpec((R, 1), lambda i: (i, 0)),
        out_shape=jax.ShapeDtypeStruct((B, 1), jnp.int32),
        compiler_params=pltpu.CompilerParams(
            dimension_semantics=("parallel",),
        ),
    )(x, w1t, w2t, w3t)

    out = pl.pallas_call(
        _mask_kernel,
        grid=(B // R,),
        in_specs=[
            pl.BlockSpec((R, Q), lambda i: (i, 0)),
            pl.BlockSpec((R, _NB), lambda i: (i, 0)),
            pl.BlockSpec((R, 1), lambda i: (i, 0)),
            pl.BlockSpec((Q, D), lambda i: (0, 0)),
        ],
        out_specs=pl.BlockSpec((R, D), lambda i: (i, 0)),
        out_shape=jax.ShapeDtypeStruct((B, D), jnp.float32),
        compiler_params=pltpu.CompilerParams(
            dimension_semantics=("parallel",),
        ),
    )(x, cdf, kvals, wct)
    return out


# speedup vs baseline: 1.0312x; 1.0312x over previous
import functools

import jax
import jax.numpy as jnp
from jax import lax
from jax.experimental import pallas as pl
from jax.experimental.pallas import tpu as pltpu
from jax.experimental.pallas import tpu_sc as plsc

_Q = 512
_NB = 128
_SHIFT = 25
_NW = 32
_CH = 8


def _monotone_key(x):
    i = lax.bitcast_convert_type(x, jnp.int32)
    flip = lax.shift_right_arithmetic(i, 31) & jnp.int32(0x7FFFFFFF)
    return i ^ flip


def _sc_hist_body(x_hbm, out_hbm, xbuf, hbuf):
    nc = 2
    wid = lax.axis_index("s") * nc + lax.axis_index("c")
    rows_per_worker = 16384 // _NW
    n_chunks = rows_per_worker // 16

    zeros16 = jnp.zeros((16,), jnp.int32)
    ones16 = jnp.ones((16,), jnp.int32)

    lane_iota = lax.iota(jnp.int32, 16)

    def chunk_body(ci, carry):
        g = wid * (rows_per_worker // 16) + ci
        pltpu.sync_copy(x_hbm.at[g], xbuf.at[:, :_Q])
        def zero_body(j, c):
            for u in range(8):
                hbuf[pl.ds((j * 8 + u) * 16, 16)] = zeros16
            return c
        carry = lax.fori_loop(0, _NB // 8, zero_body, carry)
        def hist_body(p, c):
            for u in range(8):
                xv = plsc.load_gather(xbuf, [lane_iota, jnp.full((16,), 0, jnp.int32) + (p * 8 + u)])
                key = _monotone_key(xv)
                b = lax.shift_right_arithmetic(key, _SHIFT) + jnp.int32(_NB // 2)
                plsc.addupdate_scatter(hbuf, [b * 16 + lax.iota(jnp.int32, 16)],
                                       ones16)
            return c
        carry = lax.fori_loop(0, _Q // 8, hist_body, carry)
        def scan_body(j, run):
            for u in range(8):
                run = run + hbuf[pl.ds((j * 8 + u) * 16, 16)]
                hbuf[pl.ds((j * 8 + u) * 16, 16)] = run
            return run
        lax.fori_loop(0, _NB // 8, scan_body, zeros16)
        pltpu.sync_copy(hbuf, out_hbm.at[pl.ds(g * 16 * _NB, 16 * _NB)])
        return carry

    lax.fori_loop(0, n_chunks, chunk_body, jnp.int32(0))


def _sc_hist(x):
    B = x.shape[0]
    mesh = plsc.VectorSubcoreMesh(core_axis_name="c", subcore_axis_name="s")
    kern = functools.partial(
        pl.kernel,
        mesh=mesh,
        out_type=jax.ShapeDtypeStruct((B * _NB,), jnp.int32),
        scratch_types=[
            pltpu.VMEM((16, 513), jnp.float32),
            pltpu.VMEM((_NB * 16,), jnp.int32),
        ],
        compiler_params=pltpu.CompilerParams(needs_layout_passes=False),
    )(_sc_hist_body)
    out = kern(x.reshape(B // 16, 16, _Q)).reshape(B // 16, _NB, 16)
    return out.transpose(0, 2, 1).reshape(B, _NB)


def _mlp_kernel(x_ref, w1t_ref, w2t_ref, w3t_ref, k_ref):
    xb = x_ref[...]
    h = jnp.maximum(jnp.dot(xb, w1t_ref[...], preferred_element_type=jnp.float32), 0.0)
    h = jnp.maximum(jnp.dot(h, w2t_ref[...], preferred_element_type=jnp.float32), 0.0)
    scores = jnp.dot(h, w3t_ref[...], preferred_element_type=jnp.float32)
    k_ref[...] = (jnp.argmax(scores, axis=-1).astype(jnp.int32) + 1)[:, None]


def _mask_kernel(x_ref, cdf_ref, k_ref, wct_ref, o_ref):
    xb = x_ref[...]
    R, Q = xb.shape
    k = k_ref[...]
    keys = _monotone_key(xb)

    P = cdf_ref[...]
    bstar = jnp.sum((P <= (Q - k)).astype(jnp.int32), axis=-1, keepdims=True)

    lo0 = (bstar - jnp.int32(_NB // 2)) << _SHIFT
    hi0 = jnp.where(bstar == jnp.int32(_NB - 1),
                    jnp.int32(2147483647), lo0 + jnp.int32(1 << _SHIFT))

    def body(carry):
        lo, hi = carry
        mid = (lax.shift_right_arithmetic(lo, 1)
               + lax.shift_right_arithmetic(hi, 1)
               + (lo & hi & jnp.int32(1)))
        cnt = jnp.sum((keys >= mid).astype(jnp.int32), axis=-1, keepdims=True)
        pred = cnt >= k
        lo = jnp.where(pred, mid, lo)
        hi = jnp.where(pred, hi, mid)
        return lo, hi

    carry = (lo0, hi0)
    for _ in range(_SHIFT):
        carry = body(carry)
    t = carry[0]

    gt = keys > t
    eq = keys == t
    g = jnp.sum(gt.astype(jnp.int32), axis=-1, keepdims=True)

    eq_f = eq.astype(jnp.float32)
    jj = lax.broadcasted_iota(jnp.int32, (Q, Q), 0)
    ii = lax.broadcasted_iota(jnp.int32, (Q, Q), 1)
    strict_lower = (jj < ii).astype(jnp.float32)
    prefix = jnp.dot(eq_f, strict_lower, preferred_element_type=jnp.float32)
    prefix = prefix.astype(jnp.int32)

    mask = gt | (eq & ((g + prefix) < k))
    o_ref[...] = jnp.dot(mask.astype(jnp.float32), wct_ref[...],
                         preferred_element_type=jnp.float32)


@jax.jit
def kernel(x, W1, W2, W3, Wc):
    B, Q = x.shape
    D = Wc.shape[0]
    R = 1024

    w1t = W1.T
    w2t = W2.T
    w3t = W3.T
    wct = Wc.T

    cdf = _sc_hist(x)

    kvals = pl.pallas_call(
        _mlp_kernel,
        grid=(B // R,),
        in_specs=[
            pl.BlockSpec((R, Q), lambda i: (i, 0)),
            pl.BlockSpec((Q, 2 * Q), lambda i: (0, 0)),
            pl.BlockSpec((2 * Q, Q), lambda i: (0, 0)),
            pl.BlockSpec((Q, Q), lambda i: (0, 0)),
        ],
        out_specs=pl.BlockSpec((R, 1), lambda i: (i, 0)),
        out_shape=jax.ShapeDtypeStruct((B, 1), jnp.int32),
        compiler_params=pltpu.CompilerParams(
            dimension_semantics=("parallel",),
        ),
    )(x, w1t, w2t, w3t)

    out = pl.pallas_call(
        _mask_kernel,
        grid=(B // R,),
        in_specs=[
            pl.BlockSpec((R, Q), lambda i: (i, 0)),
            pl.BlockSpec((R, _NB), lambda i: (i, 0)),
            pl.BlockSpec((R, 1), lambda i: (i, 0)),
            pl.BlockSpec((Q, D), lambda i: (0, 0)),
        ],
        out_specs=pl.BlockSpec((R, D), lambda i: (i, 0)),
        out_shape=jax.ShapeDtypeStruct((B, D), jnp.float32),
        compiler_params=pltpu.CompilerParams(
            dimension_semantics=("parallel",),
        ),
    )(x, cdf, kvals, wct)
    return out


# speedup vs baseline: 1.3573x; 1.3162x over previous
import functools

import jax
import jax.numpy as jnp
from jax import lax
from jax.experimental import pallas as pl
from jax.experimental.pallas import tpu as pltpu
from jax.experimental.pallas import tpu_sc as plsc

_Q = 512
_NB = 128
_SHIFT = 25
_NW = 32
_CH = 8


def _monotone_key(x):
    i = lax.bitcast_convert_type(x, jnp.int32)
    flip = lax.shift_right_arithmetic(i, 31) & jnp.int32(0x7FFFFFFF)
    return i ^ flip


def _sc_hist_body(x_hbm, out_hbm, xbuf, hbuf):
    nc = 2
    wid = lax.axis_index("s") * nc + lax.axis_index("c")
    rows_per_worker = 16384 // _NW
    n_chunks = rows_per_worker // 16

    zeros16 = jnp.zeros((16,), jnp.int32)
    ones16 = jnp.ones((16,), jnp.int32)

    lane_iota = lax.iota(jnp.int32, 16)

    def chunk_body(ci, carry):
        g = wid * (rows_per_worker // 16) + ci
        pltpu.sync_copy(x_hbm.at[g], xbuf.at[:, :_Q])
        @plsc.parallel_loop(0, _NB // 8, carry=carry)
        def carry(j, c):
            for u in range(8):
                hbuf[pl.ds((j * 8 + u) * 16, 16)] = zeros16
            return c
        @plsc.parallel_loop(0, _Q // 8, carry=carry)
        def carry(p, c):
            for u in range(8):
                xv = plsc.load_gather(xbuf, [lane_iota, jnp.full((16,), 0, jnp.int32) + (p * 8 + u)])
                key = _monotone_key(xv)
                b = lax.shift_right_arithmetic(key, _SHIFT) + jnp.int32(_NB // 2)
                plsc.addupdate_scatter(hbuf, [b * 16 + lax.iota(jnp.int32, 16)],
                                       ones16)
            return c
        @plsc.parallel_loop(0, _NB // 8, carry=zeros16)
        def _run(j, run):
            for u in range(8):
                run = run + hbuf[pl.ds((j * 8 + u) * 16, 16)]
                hbuf[pl.ds((j * 8 + u) * 16, 16)] = run
            return run
        pltpu.sync_copy(hbuf, out_hbm.at[pl.ds(g * 16 * _NB, 16 * _NB)])
        return carry

    lax.fori_loop(0, n_chunks, chunk_body, jnp.int32(0))


def _sc_hist(x):
    B = x.shape[0]
    mesh = plsc.VectorSubcoreMesh(core_axis_name="c", subcore_axis_name="s")
    kern = functools.partial(
        pl.kernel,
        mesh=mesh,
        out_type=jax.ShapeDtypeStruct((B * _NB,), jnp.int32),
        scratch_types=[
            pltpu.VMEM((16, 513), jnp.float32),
            pltpu.VMEM((_NB * 16,), jnp.int32),
        ],
        compiler_params=pltpu.CompilerParams(needs_layout_passes=False),
    )(_sc_hist_body)
    out = kern(x.reshape(B // 16, 16, _Q)).reshape(B // 16, _NB, 16)
    return out.transpose(0, 2, 1).reshape(B, _NB)


def _mlp_kernel(x_ref, w1t_ref, w2t_ref, w3t_ref, k_ref):
    xb = x_ref[...]
    h = jnp.maximum(jnp.dot(xb, w1t_ref[...], preferred_element_type=jnp.float32), 0.0)
    h = jnp.maximum(jnp.dot(h, w2t_ref[...], preferred_element_type=jnp.float32), 0.0)
    scores = jnp.dot(h, w3t_ref[...], preferred_element_type=jnp.float32)
    k_ref[...] = (jnp.argmax(scores, axis=-1).astype(jnp.int32) + 1)[:, None]


def _mask_kernel(x_ref, cdf_ref, k_ref, wct_ref, o_ref):
    xb = x_ref[...]
    R, Q = xb.shape
    k = k_ref[...]
    keys = _monotone_key(xb)

    P = cdf_ref[...]
    bstar = jnp.sum((P <= (Q - k)).astype(jnp.int32), axis=-1, keepdims=True)

    lo0 = (bstar - jnp.int32(_NB // 2)) << _SHIFT
    hi0 = jnp.where(bstar == jnp.int32(_NB - 1),
                    jnp.int32(2147483647), lo0 + jnp.int32(1 << _SHIFT))

    def body(carry):
        lo, hi = carry
        mid = (lax.shift_right_arithmetic(lo, 1)
               + lax.shift_right_arithmetic(hi, 1)
               + (lo & hi & jnp.int32(1)))
        cnt = jnp.sum((keys >= mid).astype(jnp.int32), axis=-1, keepdims=True)
        pred = cnt >= k
        lo = jnp.where(pred, mid, lo)
        hi = jnp.where(pred, hi, mid)
        return lo, hi

    carry = (lo0, hi0)
    for _ in range(_SHIFT):
        carry = body(carry)
    t = carry[0]

    gt = keys > t
    eq = keys == t
    g = jnp.sum(gt.astype(jnp.int32), axis=-1, keepdims=True)

    eq_f = eq.astype(jnp.float32)
    jj = lax.broadcasted_iota(jnp.int32, (Q, Q), 0)
    ii = lax.broadcasted_iota(jnp.int32, (Q, Q), 1)
    strict_lower = (jj < ii).astype(jnp.float32)
    prefix = jnp.dot(eq_f, strict_lower, preferred_element_type=jnp.float32)
    prefix = prefix.astype(jnp.int32)

    mask = gt | (eq & ((g + prefix) < k))
    o_ref[...] = jnp.dot(mask.astype(jnp.float32), wct_ref[...],
                         preferred_element_type=jnp.float32)


@jax.jit
def kernel(x, W1, W2, W3, Wc):
    B, Q = x.shape
    D = Wc.shape[0]
    R = 1024

    w1t = W1.T
    w2t = W2.T
    w3t = W3.T
    wct = Wc.T

    cdf = _sc_hist(x)

    kvals = pl.pallas_call(
        _mlp_kernel,
        grid=(B // R,),
        in_specs=[
            pl.BlockSpec((R, Q), lambda i: (i, 0)),
            pl.BlockSpec((Q, 2 * Q), lambda i: (0, 0)),
            pl.BlockSpec((2 * Q, Q), lambda i: (0, 0)),
            pl.BlockSpec((Q, Q), lambda i: (0, 0)),
        ],
        out_specs=pl.BlockSpec((R, 1), lambda i: (i, 0)),
        out_shape=jax.ShapeDtypeStruct((B, 1), jnp.int32),
        compiler_params=pltpu.CompilerParams(
            dimension_semantics=("parallel",),
        ),
    )(x, w1t, w2t, w3t)

    out = pl.pallas_call(
        _mask_kernel,
        grid=(B // R,),
        in_specs=[
            pl.BlockSpec((R, Q), lambda i: (i, 0)),
            pl.BlockSpec((R, _NB), lambda i: (i, 0)),
            pl.BlockSpec((R, 1), lambda i: (i, 0)),
            pl.BlockSpec((Q, D), lambda i: (0, 0)),
        ],
        out_specs=pl.BlockSpec((R, D), lambda i: (i, 0)),
        out_shape=jax.ShapeDtypeStruct((B, D), jnp.float32),
        compiler_params=pltpu.CompilerParams(
            dimension_semantics=("parallel",),
        ),
    )(x, cdf, kvals, wct)
    return out


# speedup vs baseline: 2.0773x; 1.5305x over previous
import functools

import jax
import jax.numpy as jnp
from jax.experimental import pallas as pl
from jax.experimental.pallas import tpu as pltpu


def _monotone_key(x):
    i = jax.lax.bitcast_convert_type(x, jnp.int32)
    flip = jax.lax.shift_right_arithmetic(i, 31) & jnp.int32(0x7FFFFFFF)
    return i ^ flip


def _fused_kernel(x_ref, w1t_ref, w2t_ref, w3t_ref, wct_ref, o_ref):
    xb = x_ref[...]
    R, Q = xb.shape

    h = jnp.maximum(jnp.dot(xb, w1t_ref[...], preferred_element_type=jnp.float32), 0.0)
    h = jnp.maximum(jnp.dot(h, w2t_ref[...], preferred_element_type=jnp.float32), 0.0)
    scores = jnp.dot(h, w3t_ref[...], preferred_element_type=jnp.float32)

    k = (jnp.argmax(scores, axis=-1).astype(jnp.int32) + 1)[:, None]

    keys = _monotone_key(xb)

    lo0 = jnp.full((R, 1), jnp.int32(-2147483648))
    hi0 = jnp.full((R, 1), jnp.int32(2147483647))

    def body(_, carry):
        lo, hi = carry
        mid = (jax.lax.shift_right_arithmetic(lo, 1)
               + jax.lax.shift_right_arithmetic(hi, 1)
               + (lo & hi & jnp.int32(1)))
        cnt = jnp.sum((keys >= mid).astype(jnp.int32), axis=-1, keepdims=True)
        pred = cnt >= k
        lo = jnp.where(pred, mid, lo)
        hi = jnp.where(pred, hi, mid)
        return lo, hi

    carry = (lo0, hi0)
    for _ in range(32):
        carry = body(0, carry)
    lo, hi = carry
    t = lo

    gt = keys > t
    eq = keys == t
    g = jnp.sum(gt.astype(jnp.int32), axis=-1, keepdims=True)

    eq_f = eq.astype(jnp.float32)
    jj = jax.lax.broadcasted_iota(jnp.int32, (Q, Q), 0)
    ii = jax.lax.broadcasted_iota(jnp.int32, (Q, Q), 1)
    strict_lower = (jj < ii).astype(jnp.float32)
    prefix = jnp.dot(eq_f, strict_lower, preferred_element_type=jnp.float32)
    prefix = prefix.astype(jnp.int32)

    mask = gt | (eq & ((g + prefix) < k))
    o_ref[...] = jnp.dot(mask.astype(jnp.float32), wct_ref[...],
                         preferred_element_type=jnp.float32)


@jax.jit
def kernel(x, W1, W2, W3, Wc):
    B, Q = x.shape
    D = Wc.shape[0]
    R = 1024

    w1t = W1.T
    w2t = W2.T
    w3t = W3.T
    wct = Wc.T

    out = pl.pallas_call(
        _fused_kernel,
        grid=(B // R,),
        in_specs=[
            pl.BlockSpec((R, Q), lambda i: (i, 0)),
            pl.BlockSpec((Q, 2 * Q), lambda i: (0, 0)),
            pl.BlockSpec((2 * Q, Q), lambda i: (0, 0)),
            pl.BlockSpec((Q, Q), lambda i: (0, 0)),
            pl.BlockSpec((Q, D), lambda i: (0, 0)),
        ],
        out_specs=pl.BlockSpec((R, D), lambda i: (i, 0)),
        out_shape=jax.ShapeDtypeStruct((B, D), jnp.float32),
        compiler_params=pltpu.CompilerParams(
            dimension_semantics=("parallel",),
        ),
    )(x, w1t, w2t, w3t, wct)
    return out
